# baseline (speedup 1.0000x reference)
import functools

import jax
import jax.numpy as jnp
from jax import lax
from jax.experimental import pallas as pl
from jax.experimental.pallas import tpu as pltpu
from jax.experimental.pallas import tpu_sc as plsc

N = 10000
E = 320000
D = 128
NLAYERS = 4
AVG_D_LOG = 3.4965
EPS = 1e-5

NC = 2
NS = 16
NW = NC * NS
KN = 320
NPAD = NW * KN
NH = NS * KN
SD = NH + 8
DUMMY = NH
EPT = E // NW
CH = 400
NCHUNK = EPT // CH
BBUF = 384
BREG = EPT + BBUF
GB = 128
BN = 256
NBH = NH // BN
NB = NPAD // BN

_mesh = plsc.VectorSubcoreMesh(core_axis_name="c", subcore_axis_name="s")
_sc_params = pltpu.CompilerParams(needs_layout_passes=False)


def _iota16():
  return lax.iota(jnp.int32, 16)


def _wid():
  return lax.axis_index("c") * NS + lax.axis_index("s")


def _mo(v, n=8):
  return pl.multiple_of(v, n)


@functools.partial(
    pl.kernel,
    out_type=[
        jax.ShapeDtypeStruct((NW * NW * BREG,), jnp.int32),
        jax.ShapeDtypeStruct((NW * NW * BREG,), jnp.int32),
        jax.ShapeDtypeStruct((NW * NW * 16,), jnp.int32),
    ],
    mesh=_mesh,
    compiler_params=_sc_params,
    scratch_types=[
        pltpu.VMEM((CH,), jnp.int32),
        pltpu.VMEM((CH,), jnp.int32),
        pltpu.VMEM((NW * BBUF,), jnp.int32),
        pltpu.VMEM((NW * BBUF,), jnp.int32),
        pltpu.VMEM((16,), jnp.int32),
    ],
)
def _bucket_kernel(src, dst, bsrc, bldst, bcnt, srcv, dstv, bufs, bufd, cntv):
  w = _wid()

  def chunk_body(ch, carry):
    cnts, wposs = carry
    base = _mo(w * EPT + ch * CH)
    pltpu.sync_copy(src.at[pl.ds(base, CH)], srcv)
    pltpu.sync_copy(dst.at[pl.ds(base, CH)], dstv)

    def vreg_body(i, carry2):
      cnts2, wposs2 = carry2
      s16 = srcv[pl.ds(i * 16, 16)]
      d16 = dstv[pl.ds(i * 16, 16)]
      b16 = d16 // KN
      l16 = d16 - b16 * KN
      new_c = []
      new_w = []
      for b in range(NW):
        cnt = cnts2[b]
        wpos = wposs2[b]
        m = b16 == b
        plsc.store_compressed(bufs.at[pl.ds(b * BBUF + cnt, 16)], s16, mask=m)
        plsc.store_compressed(bufd.at[pl.ds(b * BBUF + cnt, 16)], l16, mask=m)
        pop = plsc.all_reduce_population_count(m)
        cnt = cnt + pop[0]
        full = cnt >= 256

        @pl.when(full)
        def _():
          rbase = (b * NW + w) * BREG
          pltpu.sync_copy(bufs.at[pl.ds(b * BBUF, 256)],
                          bsrc.at[pl.ds(_mo(rbase + wpos), 256)])
          pltpu.sync_copy(bufd.at[pl.ds(b * BBUF, 256)],
                          bldst.at[pl.ds(_mo(rbase + wpos), 256)])
          rs = bufs[pl.ds(b * BBUF + 256, 16)]
          rd = bufd[pl.ds(b * BBUF + 256, 16)]
          bufs[pl.ds(b * BBUF, 16)] = rs
          bufd[pl.ds(b * BBUF, 16)] = rd

        new_c.append(jnp.where(full, cnt - 256, cnt))
        new_w.append(jnp.where(full, wpos + 256, wpos))
      return tuple(new_c), tuple(new_w)

    return lax.fori_loop(0, CH // 16, vreg_body, (cnts, wposs))

  zero = jnp.int32(0)
  cnts, wposs = lax.fori_loop(
      0, NCHUNK, chunk_body,
      (tuple(zero for _ in range(NW)), tuple(zero for _ in range(NW))))

  for b in range(NW):
    rbase = (b * NW + w) * BREG
    pltpu.sync_copy(bufs.at[pl.ds(b * BBUF, BBUF)],
                    bsrc.at[pl.ds(_mo(rbase + wposs[b]), BBUF)])
    pltpu.sync_copy(bufd.at[pl.ds(b * BBUF, BBUF)],
                    bldst.at[pl.ds(_mo(rbase + wposs[b]), BBUF)])
    cntv[pl.ds(0, 16)] = jnp.full((16,), wposs[b] + cnts[b], jnp.int32)
    pltpu.sync_copy(cntv, bcnt.at[pl.ds(_mo((b * NW + w) * 16), 16)])


@functools.partial(
    pl.kernel,
    out_type=jax.ShapeDtypeStruct((NPAD, D), jnp.float32),
    mesh=_mesh,
    compiler_params=_sc_params,
    scratch_types=[
        pltpu.VMEM((KN,), jnp.int32),
        pltpu.VMEM((KN, D), jnp.float32),
        pltpu.VMEM((KN, D), jnp.float32),
        pltpu.SemaphoreType.DMA,
    ],
)
def _embed_kernel(hidx, atab, xout, idxv, gbuf, acc, sem):
  w = _wid()
  zeros = jnp.zeros((16,), jnp.float32)

  def zero_body(r, carry):
    for g in range(D // 16):
      acc[r, pl.ds(g * 16, 16)] = zeros
    return carry

  lax.fori_loop(0, KN, zero_body, jnp.int32(0))

  for i in range(9):
    pltpu.sync_copy(hidx.at[pl.ds(_mo(i * NPAD + w * KN), KN)], idxv)
    pltpu.async_copy(atab.at[idxv], gbuf, sem).wait()

    def acc_body(r, carry):
      for g in range(D // 16):
        acc[r, pl.ds(g * 16, 16)] += gbuf[r, pl.ds(g * 16, 16)]
      return carry

    lax.fori_loop(0, KN, acc_body, jnp.int32(0))

  pltpu.sync_copy(acc, xout.at[pl.ds(_mo(w * KN), KN)])


GBA = 128


def _layer_scan(x, bsrc, bldst, bcnt, gidx, ldst, rbuf, cntv, sem,
                edge_update):
  w = _wid()

  pltpu.sync_copy(bcnt.at[pl.ds(_mo(w * (NW * 16)), NW * 16)], cntv)

  def load_group(b, rbase, gbase, rem):
    pltpu.sync_copy(bsrc.at[pl.ds(_mo(rbase + gbase), GBA)], gidx[b])
    pltpu.sync_copy(bldst.at[pl.ds(_mo(rbase + gbase), GBA)], ldst[b])
    for i in range(GBA // 16):
      lanepos = i * 16 + _iota16()
      v = gidx[b][pl.ds(i * 16, 16)]
      gidx[b][pl.ds(i * 16, 16)] = jnp.where(lanepos < rem, v, 0)
    pltpu.async_copy(x.at[gidx[b]], rbuf[b], sem)

  def wait_group(b):
    pltpu.make_async_copy(x.at[gidx[b]], rbuf[b], sem).wait()

  def scan_body(ws, carry):
    cnt = cntv[pl.ds(ws * 16, 16)][0]
    ngroups = (cnt + (GBA - 1)) // GBA
    rbase = (w * NW + ws) * BREG

    @pl.when(ngroups > 0)
    def _():
      load_group(0, rbase, 0, jnp.minimum(cnt, GBA))

    def group_body(g, carry2):
      for par in range(2):
        @pl.when((g & 1) == par)
        def _():
          wait_group(par)

          @pl.when(g + 1 < ngroups)
          def _():
            nb = (g + 1) * GBA
            load_group(1 - par, rbase, nb, jnp.minimum(cnt - nb, GBA))

          rem = jnp.minimum(cnt - g * GBA, GBA)

          def edge_body(e, carry3):
            edge_update(e, ldst[par], rbuf[par])
            return carry3

          lax.fori_loop(0, rem, edge_body, jnp.int32(0))
      return carry2

    return lax.fori_loop(0, ngroups, group_body, carry)

  lax.fori_loop(0, NW, scan_body, jnp.int32(0))


def _writeback(acc, out, w):
  pltpu.sync_copy(acc, out.at[pl.ds(_mo(w * KN), KN)])


@functools.partial(
    pl.kernel,
    out_type=[
        jax.ShapeDtypeStruct((NPAD, D), jnp.float32),
        jax.ShapeDtypeStruct((NPAD, D), jnp.float32),
        jax.ShapeDtypeStruct((NPAD,), jnp.float32),
    ],
    mesh=_mesh,
    compiler_params=_sc_params,
    scratch_types=[
        tuple(pltpu.VMEM((GBA,), jnp.int32) for _ in range(2)),
        tuple(pltpu.VMEM((GBA,), jnp.int32) for _ in range(2)),
        tuple(pltpu.VMEM((GBA, D), jnp.float32) for _ in range(2)),
        pltpu.VMEM((KN, D), jnp.float32),
        pltpu.VMEM((KN, D), jnp.float32),
        pltpu.VMEM((KN,), jnp.float32),
        pltpu.VMEM((NW * 16,), jnp.int32),
        pltpu.SemaphoreType.DMA,
    ],
)
def _seg_kernel(x, bsrc, bldst, bcnt, omax, osum, odeg,
                gidx, ldst, rbuf, amax, asum, adeg, cntv, sem):
  w = _wid()
  neg = jnp.full((16,), -1e30, jnp.float32)
  zeros = jnp.zeros((16,), jnp.float32)

  def init_body(r, carry):
    for g in range(D // 16):
      amax[r, pl.ds(g * 16, 16)] = neg
      asum[r, pl.ds(g * 16, 16)] = zeros
    return carry

  lax.fori_loop(0, KN, init_body, jnp.int32(0))
  for i in range(KN // 16):
    adeg[pl.ds(i * 16, 16)] = zeros

  def edge_update(e, ldst_b, rbuf_b):
    base16 = (e >> 4) << 4
    d16 = ldst_b[pl.ds(base16, 16)]
    db = jnp.take(d16, jnp.full((16,), e & 15, jnp.int32))
    col = _iota16()
    for g2 in range(D // 16):
      colg = g2 * 16 + col
      row = rbuf_b[e, pl.ds(g2 * 16, 16)]
      old = plsc.load_gather(amax, [db, colg])
      plsc.store_scatter(amax, [db, colg], jnp.maximum(old, row))
      old2 = plsc.load_gather(asum, [db, colg])
      plsc.store_scatter(asum, [db, colg], old2 + row)
    oldd = plsc.load_gather(adeg, [db])
    plsc.store_scatter(adeg, [db], oldd + 1.0)

  _layer_scan(x, bsrc, bldst, bcnt, gidx, ldst, rbuf, cntv, sem, edge_update)

  _writeback(amax, omax, w)
  _writeback(asum, osum, w)
  pltpu.sync_copy(adeg, odeg.at[pl.ds(_mo(w * KN), KN)])


@functools.partial(
    pl.kernel,
    out_type=[
        jax.ShapeDtypeStruct((NPAD, D), jnp.float32),
        jax.ShapeDtypeStruct((NPAD, D), jnp.float32),
    ],
    mesh=_mesh,
    compiler_params=_sc_params,
    scratch_types=[
        tuple(pltpu.VMEM((GBA,), jnp.int32) for _ in range(2)),
        tuple(pltpu.VMEM((GBA,), jnp.int32) for _ in range(2)),
        tuple(pltpu.VMEM((GBA, D), jnp.float32) for _ in range(2)),
        pltpu.VMEM((KN, D), jnp.float32),
        pltpu.VMEM((KN, D), jnp.float32),
        pltpu.VMEM((NW * 16,), jnp.int32),
        pltpu.SemaphoreType.DMA,
    ],
)
def _sq_kernel(x, bsrc, bldst, bcnt, omin, osq,
               gidx, ldst, rbuf, amin, asq, cntv, sem):
  w = _wid()
  pos = jnp.full((16,), 1e30, jnp.float32)
  zeros = jnp.zeros((16,), jnp.float32)

  def init_body(r, carry):
    for g in range(D // 16):
      amin[r, pl.ds(g * 16, 16)] = pos
      asq[r, pl.ds(g * 16, 16)] = zeros
    return carry

  lax.fori_loop(0, KN, init_body, jnp.int32(0))

  def edge_update(e, ldst_b, rbuf_b):
    base16 = (e >> 4) << 4
    d16 = ldst_b[pl.ds(base16, 16)]
    db = jnp.take(d16, jnp.full((16,), e & 15, jnp.int32))
    col = _iota16()
    for g2 in range(D // 16):
      colg = g2 * 16 + col
      row = rbuf_b[e, pl.ds(g2 * 16, 16)]
      old = plsc.load_gather(amin, [db, colg])
      plsc.store_scatter(amin, [db, colg], jnp.minimum(old, row))
      old2 = plsc.load_gather(asq, [db, colg])
      plsc.store_scatter(asq, [db, colg], old2 + row * row)

  _layer_scan(x, bsrc, bldst, bcnt, gidx, ldst, rbuf, cntv, sem, edge_update)

  _writeback(amin, omin, w)
  _writeback(asq, osq, w)


def _dense1_body(sum_ref, sq_ref, deg_ref, mx_ref, mn_ref,
                 wa_ref, wb_ref, wc_ref, b_ref,
                 hn_ref, s1_ref, s2_ref):
  i = pl.program_id(0)
  deg = deg_ref[...]
  degc = jnp.maximum(deg, 1.0)
  mean = sum_ref[...] / degc
  sq = sq_ref[...] / degc
  std = jnp.sqrt(jnp.maximum(sq - mean * mean, 0.0) + EPS)
  has = deg > 0
  mx = jnp.where(has, mx_ref[...], 0.0)
  mn = jnp.where(has, mn_ref[...], 0.0)
  agg = jnp.concatenate([mean, mx, mn, std], axis=1)
  logd = jnp.log(deg + 1.0)
  amp = logd / AVG_D_LOG
  att = AVG_D_LOG / jnp.maximum(logd, EPS)
  hn = (jnp.dot(agg, wa_ref[...], preferred_element_type=jnp.float32)
        + amp * jnp.dot(agg, wb_ref[...], preferred_element_type=jnp.float32)
        + att * jnp.dot(agg, wc_ref[...], preferred_element_type=jnp.float32)
        + b_ref[...])
  hn_ref[...] = hn
  rowid = i * BN + lax.broadcasted_iota(jnp.int32, (BN, 1), 0)
  valid = rowid < N
  hnm = jnp.where(valid, hn, 0.0)

  @pl.when(i == 0)
  def _():
    s1_ref[...] = jnp.zeros_like(s1_ref)
    s2_ref[...] = jnp.zeros_like(s2_ref)

  s1_ref[...] += jnp.sum(hnm, axis=0, keepdims=True)
  s2_ref[...] += jnp.sum(hnm * hnm, axis=0, keepdims=True)


def _dense1(ssum, ssq, deg, mx, mn, wa, wb, wc, b):
  return pl.pallas_call(
      _dense1_body,
      grid=(NB,),
      in_specs=[
          pl.BlockSpec((BN, D), lambda i: (i, 0)),
          pl.BlockSpec((BN, D), lambda i: (i, 0)),
          pl.BlockSpec((BN, 1), lambda i: (i, 0)),
          pl.BlockSpec((BN, D), lambda i: (i, 0)),
          pl.BlockSpec((BN, D), lambda i: (i, 0)),
          pl.BlockSpec((4 * D, D), lambda i: (0, 0)),
          pl.BlockSpec((4 * D, D), lambda i: (0, 0)),
          pl.BlockSpec((4 * D, D), lambda i: (0, 0)),
          pl.BlockSpec((1, D), lambda i: (0, 0)),
      ],
      out_specs=[
          pl.BlockSpec((BN, D), lambda i: (i, 0)),
          pl.BlockSpec((1, D), lambda i: (0, 0)),
          pl.BlockSpec((1, D), lambda i: (0, 0)),
      ],
      out_shape=[
          jax.ShapeDtypeStruct((NPAD, D), jnp.float32),
          jax.ShapeDtypeStruct((1, D), jnp.float32),
          jax.ShapeDtypeStruct((1, D), jnp.float32),
      ],
  )(ssum, ssq, deg, mx, mn, wa, wb, wc, b)


def _dense2_body(x_ref, hn_ref, s1_ref, s2_ref,
                 gamma_ref, beta_ref, snorm_ref, ox_ref):
  mu = s1_ref[...] / N
  var = jnp.maximum(s2_ref[...] / N - mu * mu, 0.0)
  hn = hn_ref[...]
  hnn = gamma_ref[...] * (hn - mu) / jnp.sqrt(var + EPS) + beta_ref[...]
  hnn = jnp.maximum(hnn, 0.0)
  ox_ref[...] = (x_ref[...] + hnn) * snorm_ref[...]


def _dense2(x, hn, s1, s2, gamma, beta, snorm):
  return pl.pallas_call(
      _dense2_body,
      grid=(NB,),
      in_specs=[
          pl.BlockSpec((BN, D), lambda i: (i, 0)),
          pl.BlockSpec((BN, D), lambda i: (i, 0)),
          pl.BlockSpec((1, D), lambda i: (0, 0)),
          pl.BlockSpec((1, D), lambda i: (0, 0)),
          pl.BlockSpec((1, D), lambda i: (0, 0)),
          pl.BlockSpec((1, D), lambda i: (0, 0)),
          pl.BlockSpec((BN, 1), lambda i: (i, 0)),
      ],
      out_specs=pl.BlockSpec((BN, D), lambda i: (i, 0)),
      out_shape=jax.ShapeDtypeStruct((NPAD, D), jnp.float32),
  )(x, hn, s1, s2, gamma, beta, snorm)


def _readout_body(x_ref, w1_ref, b1_ref, w2_ref, b2_ref,
                  w3_ref, b3_ref, out_ref, acc_ref):
  i = pl.program_id(0)

  @pl.when(i == 0)
  def _():
    acc_ref[...] = jnp.zeros_like(acc_ref)

  rowid = i * BN + lax.broadcasted_iota(jnp.int32, (BN, 1), 0)
  xm = jnp.where(rowid < N, x_ref[...], 0.0)
  acc_ref[...] += jnp.sum(xm, axis=0, keepdims=True)

  @pl.when(i == NB - 1)
  def _():
    hg = acc_ref[...] / N
    y = jnp.maximum(jnp.dot(hg, w1_ref[...],
                            preferred_element_type=jnp.float32) + b1_ref[...],
                    0.0)
    y = jnp.maximum(jnp.dot(y, w2_ref[...],
                            preferred_element_type=jnp.float32) + b2_ref[...],
                    0.0)
    out_ref[...] = jnp.dot(y, w3_ref[...],
                           preferred_element_type=jnp.float32) + b3_ref[...]


def _readout(x, w1, b1, w2, b2, w3, b3):
  return pl.pallas_call(
      _readout_body,
      grid=(NB,),
      in_specs=[
          pl.BlockSpec((BN, D), lambda i: (i, 0)),
          pl.BlockSpec((D, D // 2), lambda i: (0, 0)),
          pl.BlockSpec((1, D // 2), lambda i: (0, 0)),
          pl.BlockSpec((D // 2, D // 4), lambda i: (0, 0)),
          pl.BlockSpec((1, D // 4), lambda i: (0, 0)),
          pl.BlockSpec((D // 4, 1), lambda i: (0, 0)),
          pl.BlockSpec((1, 1), lambda i: (0, 0)),
      ],
      out_specs=pl.BlockSpec((1, 1), lambda i: (0, 0)),
      out_shape=jax.ShapeDtypeStruct((1, 1), jnp.float32),
      scratch_shapes=[pltpu.VMEM((1, D), jnp.float32)],
  )(x, w1, b1, w2, b2, w3, b3)


def kernel(h, edge_index, e, snorm_n, snorm_e, atom_emb, W_post, b_post,
           bn_gamma, bn_beta, W1, b1, W2, b2, W3, b3):
  src = edge_index[0].astype(jnp.int32)
  dst = edge_index[1].astype(jnp.int32)

  h32 = h.astype(jnp.int32) + 100 * jnp.arange(9, dtype=jnp.int32)[None, :]
  hidx = jnp.zeros((9, NPAD), jnp.int32).at[:, :N].set(h32.T).reshape(-1)

  atab = atom_emb.reshape(9 * 100, D)
  snorm_p = jnp.zeros((NPAD, 1), jnp.float32).at[:N].set(snorm_n)

  bsrc, bldst, bcnt = _bucket_kernel(src, dst)
  x0 = _embed_kernel(hidx, atab)

  wa_s = W_post[:, : 4 * D]
  wb_s = W_post[:, 4 * D: 8 * D]
  wc_s = W_post[:, 8 * D:]

  def layer_step(x, params):
    wa, wb, wc, bl, gamma, beta = params
    mx, ssum, deg_flat = _seg_kernel(x, bsrc, bldst, bcnt)
    mn, ssq = _sq_kernel(x, bsrc, bldst, bcnt)
    deg = deg_flat.reshape(NPAD, 1)
    hn, s1, s2 = _dense1(ssum, ssq, deg, mx, mn, wa, wb, wc, bl[None, :])
    xn = _dense2(x, hn, s1, s2, gamma[None, :], beta[None, :], snorm_p)
    return xn, None

  x, _ = lax.scan(layer_step, x0,
                  (wa_s, wb_s, wc_s, b_post, bn_gamma, bn_beta))

  return _readout(x, W1, b1[None, :], W2, b2[None, :], W3, b3[None, :])


# speedup vs baseline: 2.2214x; 2.2214x over previous
import functools

import jax
import jax.numpy as jnp
from jax import lax
from jax.experimental import pallas as pl
from jax.experimental.pallas import tpu as pltpu
from jax.experimental.pallas import tpu_sc as plsc

N = 10000
E = 320000
D = 128
NLAYERS = 4
AVG_D_LOG = 3.4965
EPS = 1e-5

NC = 2
NS = 16
NW = NC * NS
KN = 320
NPAD = NW * KN
NH = NS * KN
SD = NH + 8
DUMMY = NH
EPT = E // NW
CH = 400
NCHUNK = EPT // CH
BBUF = 384
BREG = EPT + BBUF
GB = 128
BN = 256
NBH = NH // BN
NB = NPAD // BN

_mesh = plsc.VectorSubcoreMesh(core_axis_name="c", subcore_axis_name="s")
_sc_params = pltpu.CompilerParams(needs_layout_passes=False)


def _iota16():
  return lax.iota(jnp.int32, 16)


def _wid():
  return lax.axis_index("c") * NS + lax.axis_index("s")


def _mo(v, n=8):
  return pl.multiple_of(v, n)


@functools.partial(
    pl.kernel,
    out_type=[
        jax.ShapeDtypeStruct((NW * NW * BREG,), jnp.int32),
        jax.ShapeDtypeStruct((NW * NW * BREG,), jnp.int32),
        jax.ShapeDtypeStruct((NW * NW * 16,), jnp.int32),
    ],
    mesh=_mesh,
    compiler_params=_sc_params,
    scratch_types=[
        pltpu.VMEM((CH,), jnp.int32),
        pltpu.VMEM((CH,), jnp.int32),
        pltpu.VMEM((NW * BBUF,), jnp.int32),
        pltpu.VMEM((NW * BBUF,), jnp.int32),
        pltpu.VMEM((16,), jnp.int32),
    ],
)
def _bucket_kernel(src, dst, bsrc, bldst, bcnt, srcv, dstv, bufs, bufd, cntv):
  w = _wid()

  def chunk_body(ch, carry):
    cnts, wposs = carry
    base = _mo(w * EPT + ch * CH)
    pltpu.sync_copy(src.at[pl.ds(base, CH)], srcv)
    pltpu.sync_copy(dst.at[pl.ds(base, CH)], dstv)

    def vreg_body(i, carry2):
      cnts2, wposs2 = carry2
      s16 = srcv[pl.ds(i * 16, 16)]
      d16 = dstv[pl.ds(i * 16, 16)]
      b16 = d16 // KN
      l16 = d16 - b16 * KN
      new_c = []
      new_w = []
      for b in range(NW):
        cnt = cnts2[b]
        wpos = wposs2[b]
        m = b16 == b
        plsc.store_compressed(bufs.at[pl.ds(b * BBUF + cnt, 16)], s16, mask=m)
        plsc.store_compressed(bufd.at[pl.ds(b * BBUF + cnt, 16)], l16, mask=m)
        pop = plsc.all_reduce_population_count(m)
        cnt = cnt + pop[0]
        full = cnt >= 256

        @pl.when(full)
        def _():
          rbase = (b * NW + w) * BREG
          pltpu.sync_copy(bufs.at[pl.ds(b * BBUF, 256)],
                          bsrc.at[pl.ds(_mo(rbase + wpos), 256)])
          pltpu.sync_copy(bufd.at[pl.ds(b * BBUF, 256)],
                          bldst.at[pl.ds(_mo(rbase + wpos), 256)])
          rs = bufs[pl.ds(b * BBUF + 256, 16)]
          rd = bufd[pl.ds(b * BBUF + 256, 16)]
          bufs[pl.ds(b * BBUF, 16)] = rs
          bufd[pl.ds(b * BBUF, 16)] = rd

        new_c.append(jnp.where(full, cnt - 256, cnt))
        new_w.append(jnp.where(full, wpos + 256, wpos))
      return tuple(new_c), tuple(new_w)

    return lax.fori_loop(0, CH // 16, vreg_body, (cnts, wposs))

  zero = jnp.int32(0)
  cnts, wposs = lax.fori_loop(
      0, NCHUNK, chunk_body,
      (tuple(zero for _ in range(NW)), tuple(zero for _ in range(NW))))

  for b in range(NW):
    rbase = (b * NW + w) * BREG
    pltpu.sync_copy(bufs.at[pl.ds(b * BBUF, BBUF)],
                    bsrc.at[pl.ds(_mo(rbase + wposs[b]), BBUF)])
    pltpu.sync_copy(bufd.at[pl.ds(b * BBUF, BBUF)],
                    bldst.at[pl.ds(_mo(rbase + wposs[b]), BBUF)])
    cntv[pl.ds(0, 16)] = jnp.full((16,), wposs[b] + cnts[b], jnp.int32)
    pltpu.sync_copy(cntv, bcnt.at[pl.ds(_mo((b * NW + w) * 16), 16)])


@functools.partial(
    pl.kernel,
    out_type=jax.ShapeDtypeStruct((NPAD, D), jnp.float32),
    mesh=_mesh,
    compiler_params=_sc_params,
    scratch_types=[
        pltpu.VMEM((KN,), jnp.int32),
        pltpu.VMEM((KN, D), jnp.float32),
        pltpu.VMEM((KN, D), jnp.float32),
        pltpu.SemaphoreType.DMA,
    ],
)
def _embed_kernel(hidx, atab, xout, idxv, gbuf, acc, sem):
  w = _wid()
  zeros = jnp.zeros((16,), jnp.float32)

  def zero_body(r, carry):
    for g in range(D // 16):
      acc[r, pl.ds(g * 16, 16)] = zeros
    return carry

  lax.fori_loop(0, KN, zero_body, jnp.int32(0))

  for i in range(9):
    pltpu.sync_copy(hidx.at[pl.ds(_mo(i * NPAD + w * KN), KN)], idxv)
    pltpu.async_copy(atab.at[idxv], gbuf, sem).wait()

    def acc_body(r, carry):
      for g in range(D // 16):
        acc[r, pl.ds(g * 16, 16)] += gbuf[r, pl.ds(g * 16, 16)]
      return carry

    lax.fori_loop(0, KN, acc_body, jnp.int32(0))

  pltpu.sync_copy(acc, xout.at[pl.ds(_mo(w * KN), KN)])


GBA = 64


def _layer_scan(x, bsrc, bldst, bcnt, gidx, ldst, rbuf, cntv, sem,
                edge_update):
  w = _wid()

  pltpu.sync_copy(bcnt.at[pl.ds(_mo(w * (NW * 16)), NW * 16)], cntv)

  def load_group(b, rbase, gbase, rem):
    pltpu.sync_copy(bsrc.at[pl.ds(_mo(rbase + gbase), GBA)], gidx[b])
    pltpu.sync_copy(bldst.at[pl.ds(_mo(rbase + gbase), GBA)], ldst[b])
    for i in range(GBA // 16):
      lanepos = i * 16 + _iota16()
      v = gidx[b][pl.ds(i * 16, 16)]
      gidx[b][pl.ds(i * 16, 16)] = jnp.where(lanepos < rem, v, lanepos)
    pltpu.async_copy(x.at[gidx[b]], rbuf[b], sem)

  def wait_group(b):
    pltpu.make_async_copy(x.at[gidx[b]], rbuf[b], sem).wait()

  def scan_body(ws, carry):
    cnt = cntv[pl.ds(ws * 16, 16)][0]
    ngroups = (cnt + (GBA - 1)) // GBA
    rbase = (w * NW + ws) * BREG

    @pl.when(ngroups > 0)
    def _():
      load_group(0, rbase, 0, jnp.minimum(cnt, GBA))

    def group_body(g, carry2):
      for par in range(2):
        @pl.when((g & 1) == par)
        def _():
          wait_group(par)

          @pl.when(g + 1 < ngroups)
          def _():
            nb = (g + 1) * GBA
            load_group(1 - par, rbase, nb, jnp.minimum(cnt - nb, GBA))

          rem = jnp.minimum(cnt - g * GBA, GBA)

          def edge_body(e, carry3):
            edge_update(e, ldst[par], rbuf[par])
            return carry3

          lax.fori_loop(0, rem, edge_body, jnp.int32(0))
      return carry2

    return lax.fori_loop(0, ngroups, group_body, carry)

  lax.fori_loop(0, NW, scan_body, jnp.int32(0))


def _writeback(acc, out, w):
  pltpu.sync_copy(acc, out.at[pl.ds(_mo(w * KN), KN)])


@functools.partial(
    pl.kernel,
    out_type=[
        jax.ShapeDtypeStruct((NPAD, D), jnp.float32),
        jax.ShapeDtypeStruct((NPAD, D), jnp.float32),
        jax.ShapeDtypeStruct((NPAD,), jnp.float32),
    ],
    mesh=_mesh,
    compiler_params=_sc_params,
    scratch_types=[
        tuple(pltpu.VMEM((GBA,), jnp.int32) for _ in range(2)),
        tuple(pltpu.VMEM((GBA,), jnp.int32) for _ in range(2)),
        tuple(pltpu.VMEM((GBA, D), jnp.float32) for _ in range(2)),
        pltpu.VMEM((KN, D), jnp.float32),
        pltpu.VMEM((KN, D), jnp.float32),
        pltpu.VMEM((KN,), jnp.float32),
        pltpu.VMEM((NW * 16,), jnp.int32),
        pltpu.SemaphoreType.DMA,
    ],
)
def _seg_kernel(x, bsrc, bldst, bcnt, omax, osum, odeg,
                gidx, ldst, rbuf, amax, asum, adeg, cntv, sem):
  w = _wid()
  neg = jnp.full((16,), -1e30, jnp.float32)
  zeros = jnp.zeros((16,), jnp.float32)

  def init_body(r, carry):
    for g in range(D // 16):
      amax[r, pl.ds(g * 16, 16)] = neg
      asum[r, pl.ds(g * 16, 16)] = zeros
    return carry

  lax.fori_loop(0, KN, init_body, jnp.int32(0))
  for i in range(KN // 16):
    adeg[pl.ds(i * 16, 16)] = zeros

  def edge_update(e, ldst_b, rbuf_b):
    base16 = (e >> 4) << 4
    d16 = ldst_b[pl.ds(base16, 16)]
    db = jnp.take(d16, jnp.full((16,), e & 15, jnp.int32))
    col = _iota16()
    for g2 in range(D // 16):
      colg = g2 * 16 + col
      row = rbuf_b[e, pl.ds(g2 * 16, 16)]
      old = plsc.load_gather(amax, [db, colg])
      plsc.store_scatter(amax, [db, colg], jnp.maximum(old, row))
      old2 = plsc.load_gather(asum, [db, colg])
      plsc.store_scatter(asum, [db, colg], old2 + row)
    oldd = plsc.load_gather(adeg, [db])
    plsc.store_scatter(adeg, [db], oldd + 1.0)

  _layer_scan(x, bsrc, bldst, bcnt, gidx, ldst, rbuf, cntv, sem, edge_update)

  _writeback(amax, omax, w)
  _writeback(asum, osum, w)
  pltpu.sync_copy(adeg, odeg.at[pl.ds(_mo(w * KN), KN)])


@functools.partial(
    pl.kernel,
    out_type=[
        jax.ShapeDtypeStruct((NPAD, D), jnp.float32),
        jax.ShapeDtypeStruct((NPAD, D), jnp.float32),
    ],
    mesh=_mesh,
    compiler_params=_sc_params,
    scratch_types=[
        tuple(pltpu.VMEM((GBA,), jnp.int32) for _ in range(2)),
        tuple(pltpu.VMEM((GBA,), jnp.int32) for _ in range(2)),
        tuple(pltpu.VMEM((GBA, D), jnp.float32) for _ in range(2)),
        pltpu.VMEM((KN, D), jnp.float32),
        pltpu.VMEM((KN, D), jnp.float32),
        pltpu.VMEM((NW * 16,), jnp.int32),
        pltpu.SemaphoreType.DMA,
    ],
)
def _sq_kernel(x, bsrc, bldst, bcnt, omin, osq,
               gidx, ldst, rbuf, amin, asq, cntv, sem):
  w = _wid()
  pos = jnp.full((16,), 1e30, jnp.float32)
  zeros = jnp.zeros((16,), jnp.float32)

  def init_body(r, carry):
    for g in range(D // 16):
      amin[r, pl.ds(g * 16, 16)] = pos
      asq[r, pl.ds(g * 16, 16)] = zeros
    return carry

  lax.fori_loop(0, KN, init_body, jnp.int32(0))

  def edge_update(e, ldst_b, rbuf_b):
    base16 = (e >> 4) << 4
    d16 = ldst_b[pl.ds(base16, 16)]
    db = jnp.take(d16, jnp.full((16,), e & 15, jnp.int32))
    col = _iota16()
    for g2 in range(D // 16):
      colg = g2 * 16 + col
      row = rbuf_b[e, pl.ds(g2 * 16, 16)]
      old = plsc.load_gather(amin, [db, colg])
      plsc.store_scatter(amin, [db, colg], jnp.minimum(old, row))
      old2 = plsc.load_gather(asq, [db, colg])
      plsc.store_scatter(asq, [db, colg], old2 + row * row)

  _layer_scan(x, bsrc, bldst, bcnt, gidx, ldst, rbuf, cntv, sem, edge_update)

  _writeback(amin, omin, w)
  _writeback(asq, osq, w)


def _dense1_body(sum_ref, sq_ref, deg_ref, mx_ref, mn_ref,
                 wa_ref, wb_ref, wc_ref, b_ref,
                 hn_ref, s1_ref, s2_ref):
  i = pl.program_id(0)
  deg = deg_ref[...]
  degc = jnp.maximum(deg, 1.0)
  mean = sum_ref[...] / degc
  sq = sq_ref[...] / degc
  std = jnp.sqrt(jnp.maximum(sq - mean * mean, 0.0) + EPS)
  has = deg > 0
  mx = jnp.where(has, mx_ref[...], 0.0)
  mn = jnp.where(has, mn_ref[...], 0.0)
  agg = jnp.concatenate([mean, mx, mn, std], axis=1)
  logd = jnp.log(deg + 1.0)
  amp = logd / AVG_D_LOG
  att = AVG_D_LOG / jnp.maximum(logd, EPS)
  hn = (jnp.dot(agg, wa_ref[...], preferred_element_type=jnp.float32)
        + amp * jnp.dot(agg, wb_ref[...], preferred_element_type=jnp.float32)
        + att * jnp.dot(agg, wc_ref[...], preferred_element_type=jnp.float32)
        + b_ref[...])
  hn_ref[...] = hn
  rowid = i * BN + lax.broadcasted_iota(jnp.int32, (BN, 1), 0)
  valid = rowid < N
  hnm = jnp.where(valid, hn, 0.0)

  @pl.when(i == 0)
  def _():
    s1_ref[...] = jnp.zeros_like(s1_ref)
    s2_ref[...] = jnp.zeros_like(s2_ref)

  s1_ref[...] += jnp.sum(hnm, axis=0, keepdims=True)
  s2_ref[...] += jnp.sum(hnm * hnm, axis=0, keepdims=True)


def _dense1(ssum, ssq, deg, mx, mn, wa, wb, wc, b):
  return pl.pallas_call(
      _dense1_body,
      grid=(NB,),
      in_specs=[
          pl.BlockSpec((BN, D), lambda i: (i, 0)),
          pl.BlockSpec((BN, D), lambda i: (i, 0)),
          pl.BlockSpec((BN, 1), lambda i: (i, 0)),
          pl.BlockSpec((BN, D), lambda i: (i, 0)),
          pl.BlockSpec((BN, D), lambda i: (i, 0)),
          pl.BlockSpec((4 * D, D), lambda i: (0, 0)),
          pl.BlockSpec((4 * D, D), lambda i: (0, 0)),
          pl.BlockSpec((4 * D, D), lambda i: (0, 0)),
          pl.BlockSpec((1, D), lambda i: (0, 0)),
      ],
      out_specs=[
          pl.BlockSpec((BN, D), lambda i: (i, 0)),
          pl.BlockSpec((1, D), lambda i: (0, 0)),
          pl.BlockSpec((1, D), lambda i: (0, 0)),
      ],
      out_shape=[
          jax.ShapeDtypeStruct((NPAD, D), jnp.float32),
          jax.ShapeDtypeStruct((1, D), jnp.float32),
          jax.ShapeDtypeStruct((1, D), jnp.float32),
      ],
  )(ssum, ssq, deg, mx, mn, wa, wb, wc, b)


def _dense2_body(x_ref, hn_ref, s1_ref, s2_ref,
                 gamma_ref, beta_ref, snorm_ref, ox_ref):
  mu = s1_ref[...] / N
  var = jnp.maximum(s2_ref[...] / N - mu * mu, 0.0)
  hn = hn_ref[...]
  hnn = gamma_ref[...] * (hn - mu) / jnp.sqrt(var + EPS) + beta_ref[...]
  hnn = jnp.maximum(hnn, 0.0)
  ox_ref[...] = (x_ref[...] + hnn) * snorm_ref[...]


def _dense2(x, hn, s1, s2, gamma, beta, snorm):
  return pl.pallas_call(
      _dense2_body,
      grid=(NB,),
      in_specs=[
          pl.BlockSpec((BN, D), lambda i: (i, 0)),
          pl.BlockSpec((BN, D), lambda i: (i, 0)),
          pl.BlockSpec((1, D), lambda i: (0, 0)),
          pl.BlockSpec((1, D), lambda i: (0, 0)),
          pl.BlockSpec((1, D), lambda i: (0, 0)),
          pl.BlockSpec((1, D), lambda i: (0, 0)),
          pl.BlockSpec((BN, 1), lambda i: (i, 0)),
      ],
      out_specs=pl.BlockSpec((BN, D), lambda i: (i, 0)),
      out_shape=jax.ShapeDtypeStruct((NPAD, D), jnp.float32),
  )(x, hn, s1, s2, gamma, beta, snorm)


def _readout_body(x_ref, w1_ref, b1_ref, w2_ref, b2_ref,
                  w3_ref, b3_ref, out_ref, acc_ref):
  i = pl.program_id(0)

  @pl.when(i == 0)
  def _():
    acc_ref[...] = jnp.zeros_like(acc_ref)

  rowid = i * BN + lax.broadcasted_iota(jnp.int32, (BN, 1), 0)
  xm = jnp.where(rowid < N, x_ref[...], 0.0)
  acc_ref[...] += jnp.sum(xm, axis=0, keepdims=True)

  @pl.when(i == NB - 1)
  def _():
    hg = acc_ref[...] / N
    y = jnp.maximum(jnp.dot(hg, w1_ref[...],
                            preferred_element_type=jnp.float32) + b1_ref[...],
                    0.0)
    y = jnp.maximum(jnp.dot(y, w2_ref[...],
                            preferred_element_type=jnp.float32) + b2_ref[...],
                    0.0)
    out_ref[...] = jnp.dot(y, w3_ref[...],
                           preferred_element_type=jnp.float32) + b3_ref[...]


def _readout(x, w1, b1, w2, b2, w3, b3):
  return pl.pallas_call(
      _readout_body,
      grid=(NB,),
      in_specs=[
          pl.BlockSpec((BN, D), lambda i: (i, 0)),
          pl.BlockSpec((D, D // 2), lambda i: (0, 0)),
          pl.BlockSpec((1, D // 2), lambda i: (0, 0)),
          pl.BlockSpec((D // 2, D // 4), lambda i: (0, 0)),
          pl.BlockSpec((1, D // 4), lambda i: (0, 0)),
          pl.BlockSpec((D // 4, 1), lambda i: (0, 0)),
          pl.BlockSpec((1, 1), lambda i: (0, 0)),
      ],
      out_specs=pl.BlockSpec((1, 1), lambda i: (0, 0)),
      out_shape=jax.ShapeDtypeStruct((1, 1), jnp.float32),
      scratch_shapes=[pltpu.VMEM((1, D), jnp.float32)],
  )(x, w1, b1, w2, b2, w3, b3)


def kernel(h, edge_index, e, snorm_n, snorm_e, atom_emb, W_post, b_post,
           bn_gamma, bn_beta, W1, b1, W2, b2, W3, b3):
  src = edge_index[0].astype(jnp.int32)
  dst = edge_index[1].astype(jnp.int32)

  h32 = h.astype(jnp.int32) + 100 * jnp.arange(9, dtype=jnp.int32)[None, :]
  hidx = jnp.zeros((9, NPAD), jnp.int32).at[:, :N].set(h32.T).reshape(-1)

  atab = atom_emb.reshape(9 * 100, D)
  snorm_p = jnp.zeros((NPAD, 1), jnp.float32).at[:N].set(snorm_n)

  bsrc, bldst, bcnt = _bucket_kernel(src, dst)
  x0 = _embed_kernel(hidx, atab)

  wa_s = W_post[:, : 4 * D]
  wb_s = W_post[:, 4 * D: 8 * D]
  wc_s = W_post[:, 8 * D:]

  def layer_step(x, params):
    wa, wb, wc, bl, gamma, beta = params
    mx, ssum, deg_flat = _seg_kernel(x, bsrc, bldst, bcnt)
    mn, ssq = _sq_kernel(x, bsrc, bldst, bcnt)
    deg = deg_flat.reshape(NPAD, 1)
    hn, s1, s2 = _dense1(ssum, ssq, deg, mx, mn, wa, wb, wc, bl[None, :])
    xn = _dense2(x, hn, s1, s2, gamma[None, :], beta[None, :], snorm_p)
    return xn, None

  x, _ = lax.scan(layer_step, x0,
                  (wa_s, wb_s, wc_s, b_post, bn_gamma, bn_beta))

  return _readout(x, W1, b1[None, :], W2, b2[None, :], W3, b3[None, :])
